# baseline (speedup 1.0000x reference)
import jax
import jax.numpy as jnp
from jax.experimental import pallas as pl
from jax.experimental.pallas import tpu as pltpu


def _gcn_body(x_ref, w1_ref, b1_ref, w2_ref, b2_ref, wfc_ref, bfc_ref,
              adj_t_ref, adj_b_ref, out_ref, h1_ref, s_ref):
    phase = pl.program_id(0)
    i = pl.program_id(1)
    hh = adj_t_ref.shape[0]
    bm = 2 * hh

    @pl.when((phase == 0) & (i == 0))
    def _():
        s_ref[...] = jnp.dot(
            x_ref[...], w1_ref[...], preferred_element_type=jnp.float32
        )

    @pl.when(phase == 0)
    def _():
        for k, a_ref in enumerate((adj_t_ref, adj_b_ref)):
            acc = jnp.dot(
                a_ref[...], s_ref[...], preferred_element_type=jnp.float32
            )
            h1_ref[pl.ds(i * bm + k * hh, hh), :] = jnp.maximum(
                acc + b1_ref[...], 0.0
            )

    @pl.when((phase == 1) & (i == 0))
    def _():
        s_ref[...] = jnp.dot(
            h1_ref[...], w2_ref[...], preferred_element_type=jnp.float32
        )

    @pl.when(phase == 1)
    def _():
        for k, a_ref in enumerate((adj_t_ref, adj_b_ref)):
            t = jnp.dot(
                a_ref[...], s_ref[...], preferred_element_type=jnp.float32
            )
            t = t + b2_ref[...]
            u = jnp.dot(t, wfc_ref[...], preferred_element_type=jnp.float32)
            u = u + bfc_ref[...]
            m = jnp.max(u, axis=1, keepdims=True)
            lse = jnp.log(jnp.sum(jnp.exp(u - m), axis=1, keepdims=True)) + m
            out_ref[pl.ds(k * hh, hh), :] = u - lse


def _pick_block(n):
    for bm in (400, 200, 80, 40, 16):
        if n % bm == 0:
            return bm
    return n


@jax.jit
def kernel(x, adj, W1, b1, W2, b2, Wfc, bfc):
    n, nfeat = x.shape
    nhid = W1.shape[1]
    nclass = Wfc.shape[1]
    bm = _pick_block(n)
    hh = bm // 2
    grid = (2, n // bm)

    full = lambda *s: pl.BlockSpec(s, lambda p, i: (0,) * len(s))

    out = pl.pallas_call(
        _gcn_body,
        grid=grid,
        in_specs=[
            full(n, nfeat),
            full(nfeat, nhid),
            full(1, nhid),
            full(nhid, nhid),
            full(1, nhid),
            full(nhid, nclass),
            full(1, nclass),
            pl.BlockSpec((hh, n), lambda p, i: (2 * i, 0)),
            pl.BlockSpec((hh, n), lambda p, i: (2 * i + 1, 0)),
        ],
        out_specs=pl.BlockSpec((bm, nclass), lambda p, i: (p * i, 0)),
        out_shape=jax.ShapeDtypeStruct((n, nclass), jnp.float32),
        scratch_shapes=[
            pltpu.VMEM((n, nhid), jnp.float32),
            pltpu.VMEM((n, nhid), jnp.float32),
        ],
        compiler_params=pltpu.CompilerParams(
            dimension_semantics=("arbitrary", "arbitrary"),
        ),
    )(x, W1, b1.reshape(1, nhid), W2, b2.reshape(1, nhid),
      Wfc, bfc.reshape(1, nclass), adj, adj)

    return out


# speedup vs baseline: 1.0677x; 1.0677x over previous
import jax
import jax.numpy as jnp
from jax.experimental import pallas as pl
from jax.experimental.pallas import tpu as pltpu


def _gcn_body(x_ref, w1_ref, b1_ref, w2_ref, b2_ref, wfc_ref, bfc_ref,
              adj_ref, out_ref, h1_ref, s_ref):
    phase = pl.program_id(0)
    i = pl.program_id(1)
    bm = adj_ref.shape[0]

    @pl.when((phase == 0) & (i == 0))
    def _():
        s_ref[...] = jnp.dot(
            x_ref[...], w1_ref[...], preferred_element_type=jnp.float32
        )

    @pl.when(phase == 0)
    def _():
        acc = jnp.dot(
            adj_ref[...], s_ref[...], preferred_element_type=jnp.float32
        )
        h1_ref[pl.ds(i * bm, bm), :] = jnp.maximum(acc + b1_ref[...], 0.0)

    @pl.when((phase == 1) & (i == 0))
    def _():
        s_ref[...] = jnp.dot(
            h1_ref[...], w2_ref[...], preferred_element_type=jnp.float32
        )

    @pl.when(phase == 1)
    def _():
        t = jnp.dot(
            adj_ref[...], s_ref[...], preferred_element_type=jnp.float32
        )
        t = t + b2_ref[...]
        u = jnp.dot(t, wfc_ref[...], preferred_element_type=jnp.float32)
        u = u + bfc_ref[...]
        m = jnp.max(u, axis=1, keepdims=True)
        lse = jnp.log(jnp.sum(jnp.exp(u - m), axis=1, keepdims=True)) + m
        out_ref[...] = u - lse


def _pick_block(n):
    for bm in (400, 200, 80, 40, 16, 8):
        if n % bm == 0:
            return bm
    return n


@jax.jit
def kernel(x, adj, W1, b1, W2, b2, Wfc, bfc):
    n, nfeat = x.shape
    nhid = W1.shape[1]
    nclass = Wfc.shape[1]
    bm = _pick_block(n)
    grid = (2, n // bm)

    full = lambda *s: pl.BlockSpec(s, lambda p, i: (0,) * len(s))

    out = pl.pallas_call(
        _gcn_body,
        grid=grid,
        in_specs=[
            full(n, nfeat),
            full(nfeat, nhid),
            full(1, nhid),
            full(nhid, nhid),
            full(1, nhid),
            full(nhid, nclass),
            full(1, nclass),
            pl.BlockSpec((bm, n), lambda p, i: (i, 0)),
        ],
        out_specs=pl.BlockSpec((bm, nclass), lambda p, i: (p * i, 0)),
        out_shape=jax.ShapeDtypeStruct((n, nclass), jnp.float32),
        scratch_shapes=[
            pltpu.VMEM((n, nhid), jnp.float32),
            pltpu.VMEM((n, nhid), jnp.float32),
        ],
        compiler_params=pltpu.CompilerParams(
            dimension_semantics=("arbitrary", "arbitrary"),
        ),
    )(x, W1, b1.reshape(1, nhid), W2, b2.reshape(1, nhid),
      Wfc, bfc.reshape(1, nclass), adj)

    return out
